# baseline (speedup 1.0000x reference)
import math

import jax
import jax.numpy as jnp
from jax import lax
from jax.experimental import pallas as pl
from jax.experimental.pallas import tpu as pltpu
from jax.experimental.pallas import tpu_sc as plsc

N = 10000
E = 320000
D = 128
NC = 1
NS = 16
NW = NC * NS
EH = 162304
EP = EH // NW
CE = 32
NCH = EP // CE
PAD_DST = N + 200
GROUPS = CE // 16
NBUF = 2
NIDX = 4
NP = 10240
ROWS_PER_TILE = NP // NS
DEN_R = NP // 128
SCALE = 1.0 / math.sqrt(D)

_i32 = jnp.int32
_f32 = jnp.float32


def _sc_edge_body(q_hbm, kv_hbm, srcr_hbm, dstr_hbm, zeros_hbm,
                  acc_out, den_out,
                  sidx, didx, scidx, qbuf, kvbuf, sbuf, accbuf,
                  ident, denom_v, acc_sh, den_sh, semi, semg, sems):
  cid = lax.axis_index("c")
  sid = lax.axis_index("s")
  wid = sid * NC + cid

  zbase = sid * ROWS_PER_TILE
  pltpu.sync_copy(zeros_hbm.at[pl.ds(zbase, ROWS_PER_TILE)],
                  acc_sh.at[pl.ds(zbase, ROWS_PER_TILE)])
  pltpu.sync_copy(zeros_hbm.at[pl.ds(0, DEN_R)], denom_v)

  @pl.when(sid == 0)
  def _():
    pltpu.sync_copy(zeros_hbm.at[pl.ds(0, DEN_R)], den_sh)

  for b in range(DEN_R // 16):
    ident[pl.ds(b * 16, 16)] = lax.iota(_i32, 16) + b * 16
  plsc.subcore_barrier()

  def fire_idx(ch, p):
    pltpu.async_copy(srcr_hbm.at[wid, ch], sidx.at[p], semi)
    pltpu.async_copy(dstr_hbm.at[wid, ch], didx.at[p], semi)

  def wait_idx(ch, p):
    pltpu.make_async_copy(srcr_hbm.at[wid, ch], sidx.at[p], semi).wait()
    pltpu.make_async_copy(dstr_hbm.at[wid, ch], didx.at[p], semi).wait()

  def fire_gather(pi, pd):
    pltpu.async_copy(q_hbm.at[didx.at[pi]], qbuf.at[pd], semg)
    pltpu.async_copy(kv_hbm.at[sidx.at[pi]], kvbuf.at[pd], semg)

  def wait_gather(pi, pd):
    pltpu.make_async_copy(q_hbm.at[didx.at[pi]], qbuf.at[pd], semg).wait()
    pltpu.make_async_copy(kv_hbm.at[sidx.at[pi]], kvbuf.at[pd], semg).wait()

  def fire_scatter(p):
    pltpu.async_copy(sbuf.at[p], acc_sh.at[scidx.at[p]], sems, add=True)

  def wait_scatter(p):
    pltpu.make_async_copy(sbuf.at[p], acc_sh.at[scidx.at[p]], sems).wait()

  def compute(p, pi):
    lane16 = lax.iota(_i32, 16) * 16
    for g in range(GROUPS):
      for j in range(16):
        e = g * 16 + j
        acc = qbuf[p, e, pl.ds(0, 16)] * kvbuf[p, e, pl.ds(0, 16)]
        for c in range(1, D // 16):
          acc = acc + (qbuf[p, e, pl.ds(c * 16, 16)] *
                       kvbuf[p, e, pl.ds(c * 16, 16)])
        plsc.store_scatter(accbuf, [lane16 + j], acc)
      tot = accbuf[pl.ds(0, 16)]
      for c in range(1, 16):
        tot = tot + accbuf[pl.ds(c * 16, 16)]
      w = jnp.exp(jnp.clip(tot, -60.0, 60.0))
      dstv = didx[pi, pl.ds(g * 16, 16)]
      scidx[p, pl.ds(g * 16, 16)] = dstv
      plsc.addupdate_scatter(denom_v, [dstv >> 7, dstv & 127], w)
      for j in range(16):
        e = g * 16 + j
        wj = jnp.take(w, jnp.full((16,), j, _i32))
        for c in range(D // 16):
          sbuf[p, e, pl.ds(c * 16, 16)] = (
              kvbuf[p, e, pl.ds(D + c * 16, 16)] * wj)

  pltpu.sync_copy(srcr_hbm.at[wid, 0], sidx.at[0])
  pltpu.sync_copy(dstr_hbm.at[wid, 0], didx.at[0])
  fire_gather(0, 0)
  fire_idx(1, 1)
  fire_idx(2, 2)
  fire_idx(3, 3)

  def quad_body(i, carry):
    for b4 in range(NIDX):
      ch = i * NIDX + b4
      p = b4 % 2
      pi = b4
      pn = (b4 + 1) % NIDX
      wait_gather(pi, p)
      wait_idx(ch + 1, pn)
      fire_gather(pn, 1 - p)

      @pl.when(ch >= 2)
      def _():
        wait_scatter(p)

      compute(p, pi)

      @pl.when(ch <= NCH - NIDX - 1)
      def _():
        fire_idx(ch + NIDX, pi)

      fire_scatter(p)
    return carry

  lax.fori_loop(0, (NCH - 1) // NIDX, quad_body, 0)

  wait_gather(0, 0)
  wait_scatter(0)
  compute(0, 0)
  fire_scatter(0)
  wait_scatter(1)
  wait_scatter(0)

  pltpu.sync_copy(denom_v, den_sh.at[ident], add=True)

  plsc.subcore_barrier()
  pltpu.sync_copy(acc_sh.at[pl.ds(zbase, ROWS_PER_TILE)],
                  acc_out.at[cid, pl.ds(zbase, ROWS_PER_TILE)])

  @pl.when(sid == 0)
  def _():
    pltpu.sync_copy(den_sh, den_out.at[cid])


@jax.jit
def _sc_edge(q, kv, src_rows, dst_rows, zeros_rows):
  mesh = plsc.VectorSubcoreMesh(core_axis_name="c", subcore_axis_name="s",
                                num_cores=NC)
  return pl.kernel(
      _sc_edge_body,
      out_type=[jax.ShapeDtypeStruct((NC, NP, D), _f32),
                jax.ShapeDtypeStruct((NC, DEN_R, 128), _f32)],
      mesh=mesh,
      compiler_params=pltpu.CompilerParams(needs_layout_passes=False),
      scratch_types=[
          pltpu.VMEM((NIDX, CE), _i32),
          pltpu.VMEM((NIDX, CE), _i32),
          pltpu.VMEM((NBUF, CE), _i32),
          pltpu.VMEM((NBUF, CE, D), _f32),
          pltpu.VMEM((NBUF, CE, 2 * D), _f32),
          pltpu.VMEM((NBUF, CE, D), _f32),
          pltpu.VMEM((256,), _f32),
          pltpu.VMEM((DEN_R,), _i32),
          pltpu.VMEM((DEN_R, 128), _f32),
          pltpu.VMEM_SHARED((NP, D), _f32),
          pltpu.VMEM_SHARED((DEN_R, 128), _f32),
          pltpu.SemaphoreType.DMA,
          pltpu.SemaphoreType.DMA,
          pltpu.SemaphoreType.DMA,
      ],
  )(q, kv, src_rows, dst_rows, zeros_rows)


BLK = 1000


def _tc_qkv_body(x_ref, wq, bq, wk, bk, wv, bv, q_ref, kv_ref):
  xb = x_ref[...]
  q_ref[...] = (jnp.dot(xb, wq[...], preferred_element_type=_f32)
                + bq[...]) * SCALE
  kv_ref[:, pl.ds(0, D)] = (
      jnp.dot(xb, wk[...], preferred_element_type=_f32) + bk[...])
  kv_ref[:, pl.ds(D, D)] = (
      jnp.dot(xb, wv[...], preferred_element_type=_f32) + bv[...])


@jax.jit
def _tc_qkv(x, wq, bq, wk, bk, wv, bv):
  wspec = pl.BlockSpec((D, D), lambda i: (0, 0))
  bspec = pl.BlockSpec((1, D), lambda i: (0, 0))
  return pl.pallas_call(
      _tc_qkv_body,
      grid=(N // BLK,),
      in_specs=[pl.BlockSpec((BLK, D), lambda i: (i, 0)),
                wspec, bspec, wspec, bspec, wspec, bspec],
      out_specs=[pl.BlockSpec((BLK, D), lambda i: (i, 0)),
                 pl.BlockSpec((BLK, 2 * D), lambda i: (i, 0))],
      out_shape=[jax.ShapeDtypeStruct((N, D), _f32),
                 jax.ShapeDtypeStruct((N, 2 * D), _f32)],
  )(x, wq, bq, wk, bk, wv, bv)


def _tc_mid_body(acc_ref, accb_ref, den_ref, denb_ref, x_ref,
                 ws, bs, wq, bq, wk, bk, wv, bv,
                 h_ref, q_ref, kv_ref):
  a = acc_ref[0] + accb_ref[0]
  den = den_ref[...] + denb_ref[...] + 1e-16
  h = a / den
  h = h + jnp.dot(x_ref[...], ws[...], preferred_element_type=_f32) + bs[...]
  h = jnp.maximum(h, 0.0)
  h_ref[...] = h
  q_ref[...] = (jnp.dot(h, wq[...], preferred_element_type=_f32)
                + bq[...]) * SCALE
  kv_ref[:, pl.ds(0, D)] = (
      jnp.dot(h, wk[...], preferred_element_type=_f32) + bk[...])
  kv_ref[:, pl.ds(D, D)] = (
      jnp.dot(h, wv[...], preferred_element_type=_f32) + bv[...])


@jax.jit
def _tc_mid(acc, accb, den, denb, x, ws, bs, wq, bq, wk, bk, wv, bv):
  wspec = pl.BlockSpec((D, D), lambda i: (0, 0))
  bspec = pl.BlockSpec((1, D), lambda i: (0, 0))
  return pl.pallas_call(
      _tc_mid_body,
      grid=(N // BLK,),
      in_specs=[pl.BlockSpec((NC, BLK, D), lambda i: (0, i, 0)),
                pl.BlockSpec((NC, BLK, D), lambda i: (0, i, 0)),
                pl.BlockSpec((BLK, 1), lambda i: (i, 0)),
                pl.BlockSpec((BLK, 1), lambda i: (i, 0)),
                pl.BlockSpec((BLK, D), lambda i: (i, 0)),
                wspec, bspec, wspec, bspec, wspec, bspec, wspec, bspec],
      out_specs=[pl.BlockSpec((BLK, D), lambda i: (i, 0)),
                 pl.BlockSpec((BLK, D), lambda i: (i, 0)),
                 pl.BlockSpec((BLK, 2 * D), lambda i: (i, 0))],
      out_shape=[jax.ShapeDtypeStruct((N, D), _f32),
                 jax.ShapeDtypeStruct((N, D), _f32),
                 jax.ShapeDtypeStruct((N, 2 * D), _f32)],
  )(acc, accb, den, denb, x, ws, bs, wq, bq, wk, bk, wv, bv)


def _tc_final_body(acc_ref, accb_ref, den_ref, denb_ref, h1_ref,
                   ws, bs, wd, bd, out_ref):
  a = acc_ref[0] + accb_ref[0]
  den = den_ref[...] + denb_ref[...] + 1e-16
  h = a / den
  h = h + jnp.dot(h1_ref[...], ws[...], preferred_element_type=_f32) + bs[...]
  out_ref[...] = jnp.dot(h, wd[...], preferred_element_type=_f32) + bd[...]


@jax.jit
def _tc_final(acc, accb, den, denb, h1, ws, bs, wd, bd):
  wspec = pl.BlockSpec((D, D), lambda i: (0, 0))
  bspec = pl.BlockSpec((1, D), lambda i: (0, 0))
  return pl.pallas_call(
      _tc_final_body,
      grid=(N // BLK,),
      in_specs=[pl.BlockSpec((NC, BLK, D), lambda i: (0, i, 0)),
                pl.BlockSpec((NC, BLK, D), lambda i: (0, i, 0)),
                pl.BlockSpec((BLK, 1), lambda i: (i, 0)),
                pl.BlockSpec((BLK, 1), lambda i: (i, 0)),
                pl.BlockSpec((BLK, D), lambda i: (i, 0)),
                wspec, bspec, wspec, bspec],
      out_specs=pl.BlockSpec((BLK, D), lambda i: (i, 0)),
      out_shape=jax.ShapeDtypeStruct((N, D), _f32),
  )(acc, accb, den, denb, h1, ws, bs, wd, bd)


def kernel(x, edge_index, Wq1, bq1, Wk1, bk1, Wv1, bv1, Ws1, bs1,
           Wq2, bq2, Wk2, bk2, Wv2, bv2, Ws2, bs2, Wd, bd):
  npad = 2 * EH - E
  src_p = jnp.concatenate([edge_index[0].astype(_i32),
                           jnp.zeros((npad,), _i32)])
  dst_p = jnp.concatenate([edge_index[1].astype(_i32),
                           jnp.full((npad,), PAD_DST, _i32)])
  src_rows = src_p.reshape(2, NW, NCH, CE)
  dst_rows = dst_p.reshape(2, NW, NCH, CE)
  zeros_rows = jnp.zeros((NP, D), _f32)

  b = lambda v: v.reshape(1, D)
  dn = lambda d: d[0].reshape(NP)[:N].reshape(N, 1)

  def edge_stage(q, kv):
    acc_a, den_a = _sc_edge(q, kv, src_rows[0], dst_rows[0], zeros_rows)
    acc_b, den_b = _sc_edge(q, kv, src_rows[1], dst_rows[1], zeros_rows)
    return acc_a, acc_b, dn(den_a), dn(den_b)

  q1, kv1 = _tc_qkv(x, Wq1, b(bq1), Wk1, b(bk1), Wv1, b(bv1))
  aa1, ab1, da1, db1 = edge_stage(q1, kv1)
  h1, q2, kv2 = _tc_mid(aa1, ab1, da1, db1, x, Ws1, b(bs1), Wq2, b(bq2),
                        Wk2, b(bk2), Wv2, b(bv2))
  aa2, ab2, da2, db2 = edge_stage(q2, kv2)
  return _tc_final(aa2, ab2, da2, db2, h1, Ws2, b(bs2), Wd, b(bd))


# speedup vs baseline: 1.0301x; 1.0301x over previous
import math

import jax
import numpy as np
import jax.numpy as jnp
from jax import lax
from jax.experimental import pallas as pl
from jax.experimental.pallas import tpu as pltpu
from jax.experimental.pallas import tpu_sc as plsc

N = 10000
E = 320000
D = 128
NC = 1
NS = 16
NW = NC * NS
EP = E // NW
CE = 32
NCH = EP // CE
GROUPS = CE // 16
NBUF = 2
NIDX = 4
NP = 10240
ROWS_PER_TILE = NP // NS
DEN_R = NP // 128
SCALE = 1.0 / math.sqrt(D)

_i32 = jnp.int32
_f32 = jnp.float32
_u32 = jnp.uint32
_bf16 = jnp.bfloat16


def _sc_edge_body(q_hbm, kv_hbm, srcr_hbm, dstr_hbm, zeros_hbm,
                  acc_out, den_out,
                  sidx, didx, scidx, qbuf, kvbuf, sbuf, accbuf,
                  ident, denom_v, acc_sh, den_sh, semi, semg, sems):
  cid = lax.axis_index("c")
  sid = lax.axis_index("s")
  wid = sid * NC + cid

  zbase = sid * ROWS_PER_TILE
  pltpu.sync_copy(zeros_hbm.at[pl.ds(zbase, ROWS_PER_TILE)],
                  acc_sh.at[pl.ds(zbase, ROWS_PER_TILE)])
  pltpu.sync_copy(zeros_hbm.at[pl.ds(0, DEN_R)], denom_v)

  @pl.when(sid == 0)
  def _():
    pltpu.sync_copy(zeros_hbm.at[pl.ds(0, DEN_R)], den_sh)

  for b in range(DEN_R // 16):
    ident[pl.ds(b * 16, 16)] = lax.iota(_i32, 16) + b * 16
  plsc.subcore_barrier()

  def fire_idx(ch, p):
    pltpu.async_copy(srcr_hbm.at[wid, ch], sidx.at[p], semi)
    pltpu.async_copy(dstr_hbm.at[wid, ch], didx.at[p], semi)

  def wait_idx(ch, p):
    pltpu.make_async_copy(srcr_hbm.at[wid, ch], sidx.at[p], semi).wait()
    pltpu.make_async_copy(dstr_hbm.at[wid, ch], didx.at[p], semi).wait()

  def fire_gather(pi, pd):
    pltpu.async_copy(q_hbm.at[didx.at[pi]], qbuf.at[pd], semg)
    pltpu.async_copy(kv_hbm.at[sidx.at[pi]], kvbuf.at[pd], semg)

  def wait_gather(pi, pd):
    pltpu.make_async_copy(q_hbm.at[didx.at[pi]], qbuf.at[pd], semg).wait()
    pltpu.make_async_copy(kv_hbm.at[sidx.at[pi]], kvbuf.at[pd], semg).wait()

  def fire_scatter(p):
    pltpu.async_copy(sbuf.at[p], acc_sh.at[scidx.at[p]], sems, add=True)

  def wait_scatter(p):
    pltpu.make_async_copy(sbuf.at[p], acc_sh.at[scidx.at[p]], sems).wait()

  def compute(p, pi):
    lane16 = lax.iota(_i32, 16) * 16
    himask = jnp.full((16,), 0xFFFF0000, _u32)

    def bf2f(u):
      lo = plsc.bitcast(u << 16, _f32)
      hi = plsc.bitcast(u & himask, _f32)
      return lo, hi

    for g in range(GROUPS):
      for j in range(16):
        e = g * 16 + j
        acc = jnp.zeros((16,), _f32)
        for c in range(D // 32):
          kl, kh = bf2f(kvbuf[p, e, pl.ds(c * 16, 16)])
          acc = (acc + qbuf[p, e, pl.ds(c * 32, 16)] * kl
                 + qbuf[p, e, pl.ds(c * 32 + 16, 16)] * kh)
        plsc.store_scatter(accbuf, [lane16 + j], acc)
      tot = accbuf[pl.ds(0, 16)]
      for c in range(1, 16):
        tot = tot + accbuf[pl.ds(c * 16, 16)]
      w = jnp.exp(jnp.clip(tot, -60.0, 60.0))
      dstv = didx[pi, pl.ds(g * 16, 16)]
      scidx[p, pl.ds(g * 16, 16)] = dstv
      plsc.addupdate_scatter(denom_v, [dstv >> 7, dstv & 127], w)
      for j in range(16):
        e = g * 16 + j
        wj = jnp.take(w, jnp.full((16,), j, _i32))
        for c in range(D // 32):
          vl, vh = bf2f(kvbuf[p, e, pl.ds(D // 2 + c * 16, 16)])
          sbuf[p, e, pl.ds(c * 32, 16)] = vl * wj
          sbuf[p, e, pl.ds(c * 32 + 16, 16)] = vh * wj

  pltpu.sync_copy(srcr_hbm.at[wid, 0], sidx.at[0])
  pltpu.sync_copy(dstr_hbm.at[wid, 0], didx.at[0])
  fire_gather(0, 0)
  fire_idx(1, 1)
  fire_idx(2, 2)
  fire_idx(3, 3)

  def quad_body(i, carry):
    for b4 in range(NIDX):
      ch = i * NIDX + b4
      p = b4 % 2
      pi = b4
      pn = (b4 + 1) % NIDX
      wait_gather(pi, p)
      wait_idx(ch + 1, pn)
      fire_gather(pn, 1 - p)

      @pl.when(ch >= 2)
      def _():
        wait_scatter(p)

      compute(p, pi)

      @pl.when(ch <= NCH - NIDX - 1)
      def _():
        fire_idx(ch + NIDX, pi)

      fire_scatter(p)
    return carry

  lax.fori_loop(0, (NCH - 1) // NIDX, quad_body, 0)

  wait_gather(0, 0)
  wait_scatter(0)
  compute(0, 0)
  fire_scatter(0)
  wait_scatter(1)
  wait_scatter(0)

  pltpu.sync_copy(denom_v, den_sh.at[ident], add=True)

  plsc.subcore_barrier()
  pltpu.sync_copy(acc_sh.at[pl.ds(zbase, ROWS_PER_TILE)],
                  acc_out.at[cid, pl.ds(zbase, ROWS_PER_TILE)])

  @pl.when(sid == 0)
  def _():
    pltpu.sync_copy(den_sh, den_out.at[cid])


@jax.jit
def _sc_edge(q, kv, src_rows, dst_rows, zeros_rows):
  kv = lax.bitcast_convert_type(kv.reshape(N, D, 2), _u32)
  mesh = plsc.VectorSubcoreMesh(core_axis_name="c", subcore_axis_name="s",
                                num_cores=NC)
  return pl.kernel(
      _sc_edge_body,
      out_type=[jax.ShapeDtypeStruct((NC, NP, D), _f32),
                jax.ShapeDtypeStruct((NC, DEN_R, 128), _f32)],
      mesh=mesh,
      compiler_params=pltpu.CompilerParams(needs_layout_passes=False),
      scratch_types=[
          pltpu.VMEM((NIDX, CE), _i32),
          pltpu.VMEM((NIDX, CE), _i32),
          pltpu.VMEM((NBUF, CE), _i32),
          pltpu.VMEM((NBUF, CE, D), _f32),
          pltpu.VMEM((NBUF, CE, D), _u32),
          pltpu.VMEM((NBUF, CE, D), _f32),
          pltpu.VMEM((256,), _f32),
          pltpu.VMEM((DEN_R,), _i32),
          pltpu.VMEM((DEN_R, 128), _f32),
          pltpu.VMEM_SHARED((NP, D), _f32),
          pltpu.VMEM_SHARED((DEN_R, 128), _f32),
          pltpu.SemaphoreType.DMA,
          pltpu.SemaphoreType.DMA,
          pltpu.SemaphoreType.DMA,
      ],
  )(q, kv, src_rows, dst_rows, zeros_rows)


BLK = 1000


def _tc_qkv_body(x_ref, wq, bq, wk, bk, wv, bv, q_ref, kv_ref):
  xb = x_ref[...]
  q_ref[...] = (jnp.dot(xb, wq[...], preferred_element_type=_f32)
                + bq[...]) * SCALE
  kv_ref[:, pl.ds(0, D)] = (
      jnp.dot(xb, wk[...], preferred_element_type=_f32)
      + bk[...]).astype(_bf16)
  kv_ref[:, pl.ds(D, D)] = (
      jnp.dot(xb, wv[...], preferred_element_type=_f32)
      + bv[...]).astype(_bf16)


@jax.jit
def _tc_qkv(x, wq, bq, wk, bk, wv, bv):
  wspec = pl.BlockSpec((D, D), lambda i: (0, 0))
  bspec = pl.BlockSpec((1, D), lambda i: (0, 0))
  return pl.pallas_call(
      _tc_qkv_body,
      grid=(N // BLK,),
      in_specs=[pl.BlockSpec((BLK, D), lambda i: (i, 0)),
                wspec, bspec, wspec, bspec, wspec, bspec],
      out_specs=[pl.BlockSpec((BLK, D), lambda i: (i, 0)),
                 pl.BlockSpec((BLK, 2 * D), lambda i: (i, 0))],
      out_shape=[jax.ShapeDtypeStruct((N, D), _f32),
                 jax.ShapeDtypeStruct((N, 2 * D), _bf16)],
  )(x, wq, bq, wk, bk, wv, bv)


def _tc_mid_body(acc_ref, den_ref, x_ref, ws, bs, wq, bq, wk, bk, wv, bv,
                 h_ref, q_ref, kv_ref):
  a = acc_ref[0]
  den = den_ref[...] + 1e-16
  h = a / den
  h = h + jnp.dot(x_ref[...], ws[...], preferred_element_type=_f32) + bs[...]
  h = jnp.maximum(h, 0.0)
  h_ref[...] = h
  q_ref[...] = (jnp.dot(h, wq[...], preferred_element_type=_f32)
                + bq[...]) * SCALE
  kv_ref[:, pl.ds(0, D)] = (
      jnp.dot(h, wk[...], preferred_element_type=_f32)
      + bk[...]).astype(_bf16)
  kv_ref[:, pl.ds(D, D)] = (
      jnp.dot(h, wv[...], preferred_element_type=_f32)
      + bv[...]).astype(_bf16)


@jax.jit
def _tc_mid(acc, den, x, ws, bs, wq, bq, wk, bk, wv, bv):
  wspec = pl.BlockSpec((D, D), lambda i: (0, 0))
  bspec = pl.BlockSpec((1, D), lambda i: (0, 0))
  return pl.pallas_call(
      _tc_mid_body,
      grid=(N // BLK,),
      in_specs=[pl.BlockSpec((NC, BLK, D), lambda i: (0, i, 0)),
                pl.BlockSpec((BLK, 1), lambda i: (i, 0)),
                pl.BlockSpec((BLK, D), lambda i: (i, 0)),
                wspec, bspec, wspec, bspec, wspec, bspec, wspec, bspec],
      out_specs=[pl.BlockSpec((BLK, D), lambda i: (i, 0)),
                 pl.BlockSpec((BLK, D), lambda i: (i, 0)),
                 pl.BlockSpec((BLK, 2 * D), lambda i: (i, 0))],
      out_shape=[jax.ShapeDtypeStruct((N, D), _f32),
                 jax.ShapeDtypeStruct((N, D), _f32),
                 jax.ShapeDtypeStruct((N, 2 * D), _bf16)],
  )(acc, den, x, ws, bs, wq, bq, wk, bk, wv, bv)


def _tc_final_body(acc_ref, den_ref, h1_ref, ws, bs, wd, bd, out_ref):
  a = acc_ref[0]
  den = den_ref[...] + 1e-16
  h = a / den
  h = h + jnp.dot(h1_ref[...], ws[...], preferred_element_type=_f32) + bs[...]
  out_ref[...] = jnp.dot(h, wd[...], preferred_element_type=_f32) + bd[...]


@jax.jit
def _tc_final(acc, den, h1, ws, bs, wd, bd):
  wspec = pl.BlockSpec((D, D), lambda i: (0, 0))
  bspec = pl.BlockSpec((1, D), lambda i: (0, 0))
  return pl.pallas_call(
      _tc_final_body,
      grid=(N // BLK,),
      in_specs=[pl.BlockSpec((NC, BLK, D), lambda i: (0, i, 0)),
                pl.BlockSpec((BLK, 1), lambda i: (i, 0)),
                pl.BlockSpec((BLK, D), lambda i: (i, 0)),
                wspec, bspec, wspec, bspec],
      out_specs=pl.BlockSpec((BLK, D), lambda i: (i, 0)),
      out_shape=jax.ShapeDtypeStruct((N, D), _f32),
  )(acc, den, h1, ws, bs, wd, bd)


_pos = np.arange(D)
_blk = _pos // 32
_e = _pos % 32
PERM = np.asarray(_blk * 32 + np.where(_e % 2 == 0, _e // 2, 16 + _e // 2))


def kernel(x, edge_index, Wq1, bq1, Wk1, bk1, Wv1, bv1, Ws1, bs1,
           Wq2, bq2, Wk2, bk2, Wv2, bv2, Ws2, bs2, Wd, bd):
  Wk1, bk1, Wv1, bv1 = Wk1[:, PERM], bk1[PERM], Wv1[:, PERM], bv1[PERM]
  Wk2, bk2, Wv2, bv2 = Wk2[:, PERM], bk2[PERM], Wv2[:, PERM], bv2[PERM]
  src_rows = edge_index[0].astype(_i32).reshape(NW, NCH, CE)
  dst_rows = edge_index[1].astype(_i32).reshape(NW, NCH, CE)
  zeros_rows = jnp.zeros((NP, D), _f32)

  b = lambda v: v.reshape(1, D)
  dn = lambda d: d[0].reshape(NP)[:N].reshape(N, 1)

  q1, kv1 = _tc_qkv(x, Wq1, b(bq1), Wk1, b(bk1), Wv1, b(bv1))
  acc1, den1 = _sc_edge(q1, kv1, src_rows, dst_rows, zeros_rows)
  h1, q2, kv2 = _tc_mid(acc1, dn(den1), x, Ws1, b(bs1), Wq2, b(bq2),
                        Wk2, b(bk2), Wv2, b(bv2))
  acc2, den2 = _sc_edge(q2, kv2, src_rows, dst_rows, zeros_rows)
  return _tc_final(acc2, dn(den2), h1, Ws2, b(bs2), Wd, b(bd))


# speedup vs baseline: 1.0873x; 1.0555x over previous
import math

import jax
import jax.numpy as jnp
from jax import lax
from jax.experimental import pallas as pl
from jax.experimental.pallas import tpu as pltpu
from jax.experimental.pallas import tpu_sc as plsc

N = 10000
E = 320000
D = 128
NC = 1
NS = 16
NW = NC * NS
EP = E // NW
CE = 32
NCH = EP // CE
GROUPS = CE // 16
NBUF = 2
NIDX = 4
NP = 10240
ROWS_PER_TILE = NP // NS
DEN_R = NP // 128
SCALE = 1.0 / math.sqrt(D)

_i32 = jnp.int32
_f32 = jnp.float32


def _sc_edge_body(q_hbm, kv_hbm, sd_hbm, zeros_hbm,
                  acc_out, den_out,
                  sdix, scidx, qbuf, kvbuf, sbuf, accbuf,
                  ident, denom_v, acc_sh, den_sh, semi, semg, sems):
  cid = lax.axis_index("c")
  sid = lax.axis_index("s")
  wid = sid * NC + cid

  zbase = sid * ROWS_PER_TILE
  pltpu.sync_copy(zeros_hbm.at[pl.ds(zbase, ROWS_PER_TILE)],
                  acc_sh.at[pl.ds(zbase, ROWS_PER_TILE)])
  pltpu.sync_copy(zeros_hbm.at[pl.ds(0, DEN_R)], denom_v)

  @pl.when(sid == 0)
  def _():
    pltpu.sync_copy(zeros_hbm.at[pl.ds(0, DEN_R)], den_sh)

  for b in range(DEN_R // 16):
    ident[pl.ds(b * 16, 16)] = lax.iota(_i32, 16) + b * 16
  plsc.subcore_barrier()

  def fire_idx(ch, p):
    pltpu.async_copy(sd_hbm.at[wid, ch], sdix.at[p], semi)

  def wait_idx(ch, p):
    pltpu.make_async_copy(sd_hbm.at[wid, ch], sdix.at[p], semi).wait()

  def fire_gather(pi, pd):
    pltpu.async_copy(q_hbm.at[sdix.at[pi, 1]], qbuf.at[pd], semg)
    pltpu.async_copy(kv_hbm.at[sdix.at[pi, 0]], kvbuf.at[pd], semg)

  def wait_gather(pi, pd):
    pltpu.make_async_copy(q_hbm.at[sdix.at[pi, 1]], qbuf.at[pd], semg).wait()
    pltpu.make_async_copy(kv_hbm.at[sdix.at[pi, 0]], kvbuf.at[pd], semg).wait()

  def fire_scatter(p):
    pltpu.async_copy(sbuf.at[p], acc_sh.at[scidx.at[p]], sems, add=True)

  def wait_scatter(p):
    pltpu.make_async_copy(sbuf.at[p], acc_sh.at[scidx.at[p]], sems).wait()

  def compute(p, pi):
    lane16 = lax.iota(_i32, 16) * 16
    for g in range(GROUPS):
      for j in range(16):
        e = g * 16 + j
        acc = qbuf[p, e, pl.ds(0, 16)] * kvbuf[p, e, pl.ds(0, 16)]
        for c in range(1, D // 16):
          acc = acc + (qbuf[p, e, pl.ds(c * 16, 16)] *
                       kvbuf[p, e, pl.ds(c * 16, 16)])
        plsc.store_scatter(accbuf, [lane16 + j], acc)
      tot = accbuf[pl.ds(0, 16)]
      for c in range(1, 16):
        tot = tot + accbuf[pl.ds(c * 16, 16)]
      w = jnp.exp(jnp.clip(tot, -60.0, 60.0))
      dstv = sdix[pi, 1, pl.ds(g * 16, 16)]
      scidx[p, pl.ds(g * 16, 16)] = dstv
      plsc.addupdate_scatter(denom_v, [dstv >> 7, dstv & 127], w)
      for j in range(16):
        e = g * 16 + j
        wj = jnp.take(w, jnp.full((16,), j, _i32))
        for c in range(D // 16):
          sbuf[p, e, pl.ds(c * 16, 16)] = (
              kvbuf[p, e, pl.ds(D + c * 16, 16)] * wj)

  pltpu.sync_copy(sd_hbm.at[wid, 0], sdix.at[0])
  fire_gather(0, 0)
  fire_idx(1, 1)
  fire_idx(2, 2)
  fire_idx(3, 3)

  def quad_body(i, carry):
    for b4 in range(NIDX):
      ch = i * NIDX + b4
      p = b4 % 2
      pi = b4
      pn = (b4 + 1) % NIDX
      wait_gather(pi, p)
      wait_idx(ch + 1, pn)
      fire_gather(pn, 1 - p)

      @pl.when(ch >= 2)
      def _():
        wait_scatter(p)

      compute(p, pi)

      @pl.when(ch <= NCH - NIDX - 1)
      def _():
        fire_idx(ch + NIDX, pi)

      fire_scatter(p)
    return carry

  lax.fori_loop(0, (NCH - 1) // NIDX, quad_body, 0)

  wait_gather(0, 0)
  wait_scatter(0)
  compute(0, 0)
  fire_scatter(0)
  wait_scatter(1)
  wait_scatter(0)

  pltpu.sync_copy(denom_v, den_sh.at[ident], add=True)

  plsc.subcore_barrier()
  pltpu.sync_copy(acc_sh.at[pl.ds(zbase, ROWS_PER_TILE)],
                  acc_out.at[cid, pl.ds(zbase, ROWS_PER_TILE)])

  @pl.when(sid == 0)
  def _():
    pltpu.sync_copy(den_sh, den_out.at[cid])


@jax.jit
def _sc_edge(q, kv, sd_rows, zeros_rows):
  mesh = plsc.VectorSubcoreMesh(core_axis_name="c", subcore_axis_name="s",
                                num_cores=NC)
  return pl.kernel(
      _sc_edge_body,
      out_type=[jax.ShapeDtypeStruct((NC, NP, D), _f32),
                jax.ShapeDtypeStruct((NC, DEN_R, 128), _f32)],
      mesh=mesh,
      compiler_params=pltpu.CompilerParams(needs_layout_passes=False),
      scratch_types=[
          pltpu.VMEM((NIDX, 2, CE), _i32),
          pltpu.VMEM((NBUF, CE), _i32),
          pltpu.VMEM((NBUF, CE, D), _f32),
          pltpu.VMEM((NBUF, CE, 2 * D), _f32),
          pltpu.VMEM((NBUF, CE, D), _f32),
          pltpu.VMEM((256,), _f32),
          pltpu.VMEM((DEN_R,), _i32),
          pltpu.VMEM((DEN_R, 128), _f32),
          pltpu.VMEM_SHARED((NP, D), _f32),
          pltpu.VMEM_SHARED((DEN_R, 128), _f32),
          pltpu.SemaphoreType.DMA,
          pltpu.SemaphoreType.DMA,
          pltpu.SemaphoreType.DMA,
      ],
  )(q, kv, sd_rows, zeros_rows)


BLK = 1000


def _tc_qkv_body(x_ref, wq, bq, wk, bk, wv, bv, q_ref, kv_ref):
  xb = x_ref[...]
  q_ref[...] = (jnp.dot(xb, wq[...], preferred_element_type=_f32)
                + bq[...]) * SCALE
  kv_ref[:, pl.ds(0, D)] = (
      jnp.dot(xb, wk[...], preferred_element_type=_f32) + bk[...])
  kv_ref[:, pl.ds(D, D)] = (
      jnp.dot(xb, wv[...], preferred_element_type=_f32) + bv[...])


@jax.jit
def _tc_qkv(x, wq, bq, wk, bk, wv, bv):
  wspec = pl.BlockSpec((D, D), lambda i: (0, 0))
  bspec = pl.BlockSpec((1, D), lambda i: (0, 0))
  return pl.pallas_call(
      _tc_qkv_body,
      grid=(N // BLK,),
      in_specs=[pl.BlockSpec((BLK, D), lambda i: (i, 0)),
                wspec, bspec, wspec, bspec, wspec, bspec],
      out_specs=[pl.BlockSpec((BLK, D), lambda i: (i, 0)),
                 pl.BlockSpec((BLK, 2 * D), lambda i: (i, 0))],
      out_shape=[jax.ShapeDtypeStruct((N, D), _f32),
                 jax.ShapeDtypeStruct((N, 2 * D), _f32)],
  )(x, wq, bq, wk, bk, wv, bv)


def _tc_mid_body(acc_ref, den_ref, x_ref, ws, bs, wq, bq, wk, bk, wv, bv,
                 h_ref, q_ref, kv_ref):
  a = acc_ref[0]
  den = den_ref[...] + 1e-16
  h = a / den
  h = h + jnp.dot(x_ref[...], ws[...], preferred_element_type=_f32) + bs[...]
  h = jnp.maximum(h, 0.0)
  h_ref[...] = h
  q_ref[...] = (jnp.dot(h, wq[...], preferred_element_type=_f32)
                + bq[...]) * SCALE
  kv_ref[:, pl.ds(0, D)] = (
      jnp.dot(h, wk[...], preferred_element_type=_f32) + bk[...])
  kv_ref[:, pl.ds(D, D)] = (
      jnp.dot(h, wv[...], preferred_element_type=_f32) + bv[...])


@jax.jit
def _tc_mid(acc, den, x, ws, bs, wq, bq, wk, bk, wv, bv):
  wspec = pl.BlockSpec((D, D), lambda i: (0, 0))
  bspec = pl.BlockSpec((1, D), lambda i: (0, 0))
  return pl.pallas_call(
      _tc_mid_body,
      grid=(N // BLK,),
      in_specs=[pl.BlockSpec((NC, BLK, D), lambda i: (0, i, 0)),
                pl.BlockSpec((BLK, 1), lambda i: (i, 0)),
                pl.BlockSpec((BLK, D), lambda i: (i, 0)),
                wspec, bspec, wspec, bspec, wspec, bspec, wspec, bspec],
      out_specs=[pl.BlockSpec((BLK, D), lambda i: (i, 0)),
                 pl.BlockSpec((BLK, D), lambda i: (i, 0)),
                 pl.BlockSpec((BLK, 2 * D), lambda i: (i, 0))],
      out_shape=[jax.ShapeDtypeStruct((N, D), _f32),
                 jax.ShapeDtypeStruct((N, D), _f32),
                 jax.ShapeDtypeStruct((N, 2 * D), _f32)],
  )(acc, den, x, ws, bs, wq, bq, wk, bk, wv, bv)


def _tc_final_body(acc_ref, den_ref, h1_ref, ws, bs, wd, bd, out_ref):
  a = acc_ref[0]
  den = den_ref[...] + 1e-16
  h = a / den
  h = h + jnp.dot(h1_ref[...], ws[...], preferred_element_type=_f32) + bs[...]
  out_ref[...] = jnp.dot(h, wd[...], preferred_element_type=_f32) + bd[...]


@jax.jit
def _tc_final(acc, den, h1, ws, bs, wd, bd):
  wspec = pl.BlockSpec((D, D), lambda i: (0, 0))
  bspec = pl.BlockSpec((1, D), lambda i: (0, 0))
  return pl.pallas_call(
      _tc_final_body,
      grid=(N // BLK,),
      in_specs=[pl.BlockSpec((NC, BLK, D), lambda i: (0, i, 0)),
                pl.BlockSpec((BLK, 1), lambda i: (i, 0)),
                pl.BlockSpec((BLK, D), lambda i: (i, 0)),
                wspec, bspec, wspec, bspec],
      out_specs=pl.BlockSpec((BLK, D), lambda i: (i, 0)),
      out_shape=jax.ShapeDtypeStruct((N, D), _f32),
  )(acc, den, h1, ws, bs, wd, bd)


def kernel(x, edge_index, Wq1, bq1, Wk1, bk1, Wv1, bv1, Ws1, bs1,
           Wq2, bq2, Wk2, bk2, Wv2, bv2, Ws2, bs2, Wd, bd):
  sd_rows = jnp.stack([edge_index[0].astype(_i32).reshape(NW, NCH, CE),
                       edge_index[1].astype(_i32).reshape(NW, NCH, CE)],
                      axis=2)
  zeros_rows = jnp.zeros((NP, D), _f32)

  b = lambda v: v.reshape(1, D)
  dn = lambda d: d[0].reshape(NP)[:N].reshape(N, 1)

  q1, kv1 = _tc_qkv(x, Wq1, b(bq1), Wk1, b(bk1), Wv1, b(bv1))
  acc1, den1 = _sc_edge(q1, kv1, sd_rows, zeros_rows)
  h1, q2, kv2 = _tc_mid(acc1, dn(den1), x, Ws1, b(bs1), Wq2, b(bq2),
                        Wk2, b(bk2), Wv2, b(bv2))
  acc2, den2 = _sc_edge(q2, kv2, sd_rows, zeros_rows)
  return _tc_final(acc2, dn(den2), h1, Ws2, b(bs2), Wd, b(bd))


# speedup vs baseline: 1.1113x; 1.0221x over previous
import math

import jax
import jax.numpy as jnp
from jax import lax
from jax.experimental import pallas as pl
from jax.experimental.pallas import tpu as pltpu
from jax.experimental.pallas import tpu_sc as plsc

N = 10000
E = 320000
D = 128
NC = 1
NS = 16
NW = NC * NS
EP = E // NW
CE = 32
NCH = EP // CE
GROUPS = CE // 16
NBUF = 2
NIDX = 4
NP = 10240
ROWS_PER_TILE = NP // NS
DEN_R = NP // 128
SCALE = 1.0 / math.sqrt(D)

_i32 = jnp.int32
_f32 = jnp.float32


def _sc_edge_body(q_hbm, kv_hbm, srcr_hbm, dstr_hbm, zeros_hbm,
                  acc_out, den_out,
                  sidx, didx, scidx, qbuf, kvbuf, sbuf, accbuf,
                  ident, denom_v, acc_sh, den_sh, semi, semg, sems):
  cid = lax.axis_index("c")
  sid = lax.axis_index("s")
  wid = sid * NC + cid

  zbase = sid * ROWS_PER_TILE
  pltpu.sync_copy(zeros_hbm.at[pl.ds(zbase, ROWS_PER_TILE)],
                  acc_sh.at[pl.ds(zbase, ROWS_PER_TILE)])
  pltpu.sync_copy(zeros_hbm.at[pl.ds(0, DEN_R)], denom_v)

  @pl.when(sid == 0)
  def _():
    pltpu.sync_copy(zeros_hbm.at[pl.ds(0, DEN_R)], den_sh)

  for b in range(DEN_R // 16):
    ident[pl.ds(b * 16, 16)] = lax.iota(_i32, 16) + b * 16
  plsc.subcore_barrier()

  def fire_idx(ch, p):
    pltpu.async_copy(srcr_hbm.at[wid, ch], sidx.at[p], semi)
    pltpu.async_copy(dstr_hbm.at[wid, ch], didx.at[p], semi)

  def wait_idx(ch, p):
    pltpu.make_async_copy(srcr_hbm.at[wid, ch], sidx.at[p], semi).wait()
    pltpu.make_async_copy(dstr_hbm.at[wid, ch], didx.at[p], semi).wait()

  def fire_gather(pi, pd):
    pltpu.async_copy(q_hbm.at[didx.at[pi]], qbuf.at[pd], semg)
    pltpu.async_copy(kv_hbm.at[sidx.at[pi]], kvbuf.at[pd], semg)

  def wait_gather(pi, pd):
    pltpu.make_async_copy(q_hbm.at[didx.at[pi]], qbuf.at[pd], semg).wait()
    pltpu.make_async_copy(kv_hbm.at[sidx.at[pi]], kvbuf.at[pd], semg).wait()

  def fire_scatter(p):
    pltpu.async_copy(sbuf.at[p], acc_sh.at[scidx.at[p]], sems, add=True)

  def wait_scatter(p):
    pltpu.make_async_copy(sbuf.at[p], acc_sh.at[scidx.at[p]], sems).wait()

  def compute(p, pi):
    lane16 = lax.iota(_i32, 16) * 16
    for g in range(GROUPS):
      for j in range(16):
        e = g * 16 + j
        acc = qbuf[p, e, pl.ds(0, 16)] * kvbuf[p, e, pl.ds(0, 16)]
        for c in range(1, D // 16):
          acc = acc + (qbuf[p, e, pl.ds(c * 16, 16)] *
                       kvbuf[p, e, pl.ds(c * 16, 16)])
        plsc.store_scatter(accbuf, [lane16 + j], acc)
      tot = accbuf[pl.ds(0, 16)]
      for c in range(1, 16):
        tot = tot + accbuf[pl.ds(c * 16, 16)]
      w = jnp.exp(jnp.clip(tot, -60.0, 60.0))
      dstv = didx[pi, pl.ds(g * 16, 16)]
      scidx[p, pl.ds(g * 16, 16)] = dstv
      plsc.addupdate_scatter(denom_v, [dstv >> 7, dstv & 127], w)
      for j in range(16):
        e = g * 16 + j
        wj = jnp.take(w, jnp.full((16,), j, _i32))
        for c in range(D // 16):
          sbuf[p, e, pl.ds(c * 16, 16)] = (
              kvbuf[p, e, pl.ds(D + c * 16, 16)] * wj)

  pltpu.sync_copy(srcr_hbm.at[wid, 0], sidx.at[0])
  pltpu.sync_copy(dstr_hbm.at[wid, 0], didx.at[0])
  fire_gather(0, 0)
  fire_idx(1, 1)
  fire_idx(2, 2)
  fire_idx(3, 3)

  def quad_body(i, carry):
    for b4 in range(NIDX):
      ch = i * NIDX + b4
      p = b4 % 2
      pi = b4
      pn = (b4 + 1) % NIDX
      wait_gather(pi, p)
      wait_idx(ch + 1, pn)
      fire_gather(pn, 1 - p)

      @pl.when(ch >= 2)
      def _():
        wait_scatter(p)

      compute(p, pi)

      @pl.when(ch <= NCH - NIDX - 1)
      def _():
        fire_idx(ch + NIDX, pi)

      fire_scatter(p)
    return carry

  lax.fori_loop(0, (NCH - 1) // NIDX, quad_body, 0)

  wait_gather(0, 0)
  wait_scatter(0)
  compute(0, 0)
  fire_scatter(0)
  wait_scatter(1)
  wait_scatter(0)

  pltpu.sync_copy(denom_v, den_sh.at[ident], add=True)

  plsc.subcore_barrier()
  pltpu.sync_copy(acc_sh.at[pl.ds(zbase, ROWS_PER_TILE)],
                  acc_out.at[cid, pl.ds(zbase, ROWS_PER_TILE)])

  @pl.when(sid == 0)
  def _():
    pltpu.sync_copy(den_sh, den_out.at[cid])


@jax.jit
def _sc_edge(q, kv, src_rows, dst_rows, zeros_rows):
  mesh = plsc.VectorSubcoreMesh(core_axis_name="c", subcore_axis_name="s",
                                num_cores=NC)
  return pl.kernel(
      _sc_edge_body,
      out_type=[jax.ShapeDtypeStruct((NC, NP, D), _f32),
                jax.ShapeDtypeStruct((NC, DEN_R, 128), _f32)],
      mesh=mesh,
      compiler_params=pltpu.CompilerParams(needs_layout_passes=False),
      scratch_types=[
          pltpu.VMEM((NIDX, CE), _i32),
          pltpu.VMEM((NIDX, CE), _i32),
          pltpu.VMEM((NBUF, CE), _i32),
          pltpu.VMEM((NBUF, CE, D), _f32),
          pltpu.VMEM((NBUF, CE, 2 * D), _f32),
          pltpu.VMEM((NBUF, CE, D), _f32),
          pltpu.VMEM((256,), _f32),
          pltpu.VMEM((DEN_R,), _i32),
          pltpu.VMEM((DEN_R, 128), _f32),
          pltpu.VMEM_SHARED((NP, D), _f32),
          pltpu.VMEM_SHARED((DEN_R, 128), _f32),
          pltpu.SemaphoreType.DMA,
          pltpu.SemaphoreType.DMA,
          pltpu.SemaphoreType.DMA,
      ],
  )(q, kv, src_rows, dst_rows, zeros_rows)


BLK = 1000


def _tc_qkv_body(x_ref, wq, bq, wk, bk, wv, bv, q_ref, kv_ref):
  xb = x_ref[...]
  q_ref[...] = (jnp.dot(xb, wq[...], preferred_element_type=_f32)
                + bq[...]) * SCALE
  kv_ref[:, pl.ds(0, D)] = (
      jnp.dot(xb, wk[...], preferred_element_type=_f32) + bk[...])
  kv_ref[:, pl.ds(D, D)] = (
      jnp.dot(xb, wv[...], preferred_element_type=_f32) + bv[...])


@jax.jit
def _tc_qkv(x, wq, bq, wk, bk, wv, bv):
  wspec = pl.BlockSpec((D, D), lambda i: (0, 0))
  bspec = pl.BlockSpec((1, D), lambda i: (0, 0))
  return pl.pallas_call(
      _tc_qkv_body,
      grid=(N // BLK,),
      in_specs=[pl.BlockSpec((BLK, D), lambda i: (i, 0)),
                wspec, bspec, wspec, bspec, wspec, bspec],
      out_specs=[pl.BlockSpec((BLK, D), lambda i: (i, 0)),
                 pl.BlockSpec((BLK, 2 * D), lambda i: (i, 0))],
      out_shape=[jax.ShapeDtypeStruct((N, D), _f32),
                 jax.ShapeDtypeStruct((N, 2 * D), _f32)],
  )(x, wq, bq, wk, bk, wv, bv)


def _tc_mid_body(acc_ref, den_ref, x_ref, ws, bs, wq, bq, wk, bk, wv, bv,
                 h_ref, q_ref, kv_ref):
  a = acc_ref[0]
  den = den_ref[...] + 1e-16
  h = a / den
  h = h + jnp.dot(x_ref[...], ws[...], preferred_element_type=_f32) + bs[...]
  h = jnp.maximum(h, 0.0)
  h_ref[...] = h
  q_ref[...] = (jnp.dot(h, wq[...], preferred_element_type=_f32)
                + bq[...]) * SCALE
  kv_ref[:, pl.ds(0, D)] = (
      jnp.dot(h, wk[...], preferred_element_type=_f32) + bk[...])
  kv_ref[:, pl.ds(D, D)] = (
      jnp.dot(h, wv[...], preferred_element_type=_f32) + bv[...])


@jax.jit
def _tc_mid(acc, den, x, ws, bs, wq, bq, wk, bk, wv, bv):
  wspec = pl.BlockSpec((D, D), lambda i: (0, 0))
  bspec = pl.BlockSpec((1, D), lambda i: (0, 0))
  return pl.pallas_call(
      _tc_mid_body,
      grid=(N // BLK,),
      in_specs=[pl.BlockSpec((NC, BLK, D), lambda i: (0, i, 0)),
                pl.BlockSpec((BLK, 1), lambda i: (i, 0)),
                pl.BlockSpec((BLK, D), lambda i: (i, 0)),
                wspec, bspec, wspec, bspec, wspec, bspec, wspec, bspec],
      out_specs=[pl.BlockSpec((BLK, D), lambda i: (i, 0)),
                 pl.BlockSpec((BLK, D), lambda i: (i, 0)),
                 pl.BlockSpec((BLK, 2 * D), lambda i: (i, 0))],
      out_shape=[jax.ShapeDtypeStruct((N, D), _f32),
                 jax.ShapeDtypeStruct((N, D), _f32),
                 jax.ShapeDtypeStruct((N, 2 * D), _f32)],
  )(acc, den, x, ws, bs, wq, bq, wk, bk, wv, bv)


def _tc_final_body(acc_ref, den_ref, h1_ref, ws, bs, wd, bd, out_ref):
  a = acc_ref[0]
  den = den_ref[...] + 1e-16
  h = a / den
  h = h + jnp.dot(h1_ref[...], ws[...], preferred_element_type=_f32) + bs[...]
  out_ref[...] = jnp.dot(h, wd[...], preferred_element_type=_f32) + bd[...]


@jax.jit
def _tc_final(acc, den, h1, ws, bs, wd, bd):
  wspec = pl.BlockSpec((D, D), lambda i: (0, 0))
  bspec = pl.BlockSpec((1, D), lambda i: (0, 0))
  return pl.pallas_call(
      _tc_final_body,
      grid=(N // BLK,),
      in_specs=[pl.BlockSpec((NC, BLK, D), lambda i: (0, i, 0)),
                pl.BlockSpec((BLK, 1), lambda i: (i, 0)),
                pl.BlockSpec((BLK, D), lambda i: (i, 0)),
                wspec, bspec, wspec, bspec],
      out_specs=pl.BlockSpec((BLK, D), lambda i: (i, 0)),
      out_shape=jax.ShapeDtypeStruct((N, D), _f32),
  )(acc, den, h1, ws, bs, wd, bd)


def kernel(x, edge_index, Wq1, bq1, Wk1, bk1, Wv1, bv1, Ws1, bs1,
           Wq2, bq2, Wk2, bk2, Wv2, bv2, Ws2, bs2, Wd, bd):
  src_rows = edge_index[0].astype(_i32).reshape(NW, NCH, CE)
  dst_rows = edge_index[1].astype(_i32).reshape(NW, NCH, CE)
  zeros_rows = jnp.zeros((NP, D), _f32)

  b = lambda v: v.reshape(1, D)
  dn = lambda d: d[0].reshape(NP)[:N].reshape(N, 1)

  q1, kv1 = _tc_qkv(x, Wq1, b(bq1), Wk1, b(bk1), Wv1, b(bv1))
  acc1, den1 = _sc_edge(q1, kv1, src_rows, dst_rows, zeros_rows)
  h1, q2, kv2 = _tc_mid(acc1, dn(den1), x, Ws1, b(bs1), Wq2, b(bq2),
                        Wk2, b(bk2), Wv2, b(bv2))
  acc2, den2 = _sc_edge(q2, kv2, src_rows, dst_rows, zeros_rows)
  return _tc_final(acc2, dn(den2), h1, Ws2, b(bs2), Wd, b(bd))
